# baseline (speedup 1.0000x reference)
import functools

import jax
import jax.numpy as jnp
import numpy as np
from jax import lax
from jax.experimental import pallas as pl
from jax.experimental.pallas import tpu as pltpu

CI = 64
CO = 3
K = 7
TH = 4
TW = 4
COP = 8
QP = 8
TC = 12
NM = QP * 2 * 2 * TW * COP


def _pack_wk(weight):
    kh = np.array([[5 - 2 * j for j in range(TH)],
                   [6 - 2 * j for j in range(TH)]])
    kw = np.array([[5 - 2 * i for i in range(TW)],
                   [6 - 2 * i for i in range(TW)]])
    valid = ((kh[:, None, None, :] >= 0) & (kh[:, None, None, :] < K) &
             (kw[None, :, :, None] >= 0))
    khc, kwc = np.clip(kh, 0, K - 1), np.clip(kw, 0, K - 1)
    w = jnp.asarray(weight, jnp.float32)
    g = w[:, :, khc[:, None, None, :, None], kwc[None, :, :, None, None]]
    g = g.reshape(CI, CO, 2, 2, TW, TH) * valid[None, None]
    g = jnp.transpose(g, (2, 3, 4, 1, 5, 0))
    g = jnp.pad(g, ((0, 0), (0, 0), (0, 0), (0, COP - CO), (0, 0), (0, 0)))
    g2 = jnp.stack([jnp.pad(g, ((0, 0),) * 4 + ((qq, TC - TH - qq), (0, 0)))
                    for qq in range(QP)], axis=0)
    return g2.reshape(NM, TC * CI).astype(jnp.bfloat16)


def _body(w_ref, x_ref, o_ref, *, qh, qw):
    def compute_p(t):
        chunk = x_ref[pl.ds(QP * t, TC)].reshape(TC * CI, -1)
        return jnp.dot(w_ref[...], chunk,
                       preferred_element_type=jnp.float32)

    def combine_store(t, p):
        for qq in range(QP):
            for a in range(2):
                for b in range(2):
                    g = qq * 16 + (a * 2 + b) * TW
                    s = jnp.zeros((COP, qw), jnp.float32)
                    for i in range(TW):
                        s = s + p[(g + i) * COP:(g + i + 1) * COP, i:i + qw]
                    o_ref[b, :, 2 * QP * t + 2 * qq + a, :] = s[:CO]

    def step(t, p_prev):
        p_new = compute_p(t)
        combine_store(t - 1, p_prev)
        return p_new

    nt = qh // QP
    p_last = lax.fori_loop(1, nt, step, compute_p(0))
    combine_store(nt - 1, p_last)


def kernel(x_nchw, weight):
    n, ci, h, w = x_nchw.shape
    assert ci == CI and h % QP == 0
    oh, ow = 2 * h - 1, 2 * w - 1
    hp, wip = h + 4, w + 4

    xt = jnp.transpose(x_nchw, (0, 2, 1, 3)).astype(jnp.bfloat16)
    xp = jnp.pad(xt, ((0, 0), (1, 3), (0, 0), (1, 3)))
    wk = _pack_wk(weight)

    body = functools.partial(_body, qh=h, qw=w)
    out = pl.pallas_call(
        body,
        out_shape=jax.ShapeDtypeStruct((n, 2, CO, 2 * h, w), jnp.float32),
        grid=(n,),
        in_specs=[
            pl.BlockSpec((NM, TC * CI), lambda b: (0, 0)),
            pl.BlockSpec((None, hp, CI, wip), lambda b: (b, 0, 0, 0)),
        ],
        out_specs=pl.BlockSpec((None, 2, CO, 2 * h, w),
                               lambda b: (b, 0, 0, 0, 0)),
        compiler_params=pltpu.CompilerParams(
            dimension_semantics=("parallel",),
            vmem_limit_bytes=64 * 1024 * 1024),
    )(wk, xp)

    y = jnp.transpose(out, (0, 2, 3, 4, 1)).reshape(n, CO, 2 * h, 2 * w)
    return y[:, :, :oh, :ow]


# speedup vs baseline: 1.0016x; 1.0016x over previous
import functools

import jax
import jax.numpy as jnp
import numpy as np
from jax import lax
from jax.experimental import pallas as pl
from jax.experimental.pallas import tpu as pltpu

CI = 64
CO = 3
K = 7
TH = 4
TW = 4
COP = 8
QP = 4
TC = 8
NM = QP * 2 * 2 * TW * COP


def _pack_wk(weight):
    kh = np.array([[5 - 2 * j for j in range(TH)],
                   [6 - 2 * j for j in range(TH)]])
    kw = np.array([[5 - 2 * i for i in range(TW)],
                   [6 - 2 * i for i in range(TW)]])
    valid = ((kh[:, None, None, :] >= 0) & (kh[:, None, None, :] < K) &
             (kw[None, :, :, None] >= 0))
    khc, kwc = np.clip(kh, 0, K - 1), np.clip(kw, 0, K - 1)
    w = jnp.asarray(weight, jnp.float32)
    g = w[:, :, khc[:, None, None, :, None], kwc[None, :, :, None, None]]
    g = g.reshape(CI, CO, 2, 2, TW, TH) * valid[None, None]
    g = jnp.transpose(g, (2, 3, 4, 1, 5, 0))
    g = jnp.pad(g, ((0, 0), (0, 0), (0, 0), (0, COP - CO), (0, 0), (0, 0)))
    g2 = jnp.stack([jnp.pad(g, ((0, 0),) * 4 + ((qq, TC - TH - qq), (0, 0)))
                    for qq in range(QP)], axis=0)
    return g2.reshape(NM, TC * CI).astype(jnp.bfloat16)


def _body(w_ref, x_ref, o_ref, *, qh, qw):
    def compute_p(t):
        chunk = x_ref[pl.ds(QP * t, TC)].reshape(TC * CI, -1)
        return jnp.dot(w_ref[...], chunk,
                       preferred_element_type=jnp.float32)

    def combine_store(t, p):
        for qq in range(QP):
            for a in range(2):
                for b in range(2):
                    g = qq * 16 + (a * 2 + b) * TW
                    s = jnp.zeros((COP, qw), jnp.float32)
                    for i in range(TW):
                        s = s + p[(g + i) * COP:(g + i + 1) * COP, i:i + qw]
                    o_ref[b, :, 2 * QP * t + 2 * qq + a, :] = s[:CO]

    def step(t, p_prev):
        p_new = compute_p(t)
        combine_store(t - 1, p_prev)
        return p_new

    nt = qh // QP
    p_last = lax.fori_loop(1, nt, step, compute_p(0))
    combine_store(nt - 1, p_last)


def kernel(x_nchw, weight):
    n, ci, h, w = x_nchw.shape
    assert ci == CI and h % QP == 0
    oh, ow = 2 * h - 1, 2 * w - 1
    hp, wip = h + 4, w + 4

    xt = jnp.transpose(x_nchw, (0, 2, 1, 3)).astype(jnp.bfloat16)
    xp = jnp.pad(xt, ((0, 0), (1, 3), (0, 0), (1, 3)))
    wk = _pack_wk(weight)

    body = functools.partial(_body, qh=h, qw=w)
    out = pl.pallas_call(
        body,
        out_shape=jax.ShapeDtypeStruct((n, 2, CO, 2 * h, w), jnp.float32),
        grid=(n,),
        in_specs=[
            pl.BlockSpec((NM, TC * CI), lambda b: (0, 0)),
            pl.BlockSpec((None, hp, CI, wip), lambda b: (b, 0, 0, 0)),
        ],
        out_specs=pl.BlockSpec((None, 2, CO, 2 * h, w),
                               lambda b: (b, 0, 0, 0, 0)),
        compiler_params=pltpu.CompilerParams(
            dimension_semantics=("parallel",),
            vmem_limit_bytes=64 * 1024 * 1024),
    )(wk, xp)

    y = jnp.transpose(out, (0, 2, 3, 4, 1)).reshape(n, CO, 2 * h, 2 * w)
    return y[:, :, :oh, :ow]
